# Initial kernel scaffold; baseline (speedup 1.0000x reference)
import jax
import jax.numpy as jnp
from jax.experimental import pallas as pl


def kernel(grad_pred, x_t, grad_weight, Patchs_idx):
    raise NotImplementedError("write your pallas kernel here")


# speedup vs baseline: 3.0201x; 3.0201x over previous
import functools

import jax
import jax.numpy as jnp
from jax import lax
from jax.experimental import pallas as pl
from jax.experimental.pallas import tpu as pltpu
from jax.experimental.pallas import tpu_sc as plsc

B, N, K, D = 4, 50000, 32, 3
E = N * K
NC, NT, L = 2, 16, 16
GROUP = 128
CH = 1024
NGRP = CH * 4 // GROUP
EPT = -(-E // (NT * CH)) * CH
EP = EPT * NT
NCHUNK = EPT // CH
NPR = 3128
NACC = NPR * NT
ZLEN = NPR * 4
NBL = ZLEN + 32
OPT = NPR * D
OPAD = OPT + 8
NVAL = CH * 4 // L
NOUT = OPAD // L


def _sc_body(gp_hbm, w_hbm, idx_hbm, out_hbm,
             gp_v, w_v, idxc_v, val_v, idx4_v, nb_v, ob_v, acc0, acc1, sem):
    c = lax.axis_index("c")
    s = lax.axis_index("s")

    lane = lax.iota(jnp.int32, 16)
    sub = lane >> 2
    d = lane & 3
    d3 = d == 3
    gp_pat = 3 * sub + jnp.where(d3, 0, d)
    one16 = jnp.ones((16,), jnp.float32)

    zero16 = jnp.zeros((16,), jnp.float32)

    def zq(q, _):
        nb_v[pl.ds(16 * q, 16)] = zero16
        return _
    lax.fori_loop(0, ZLEN // 16, zq, None)
    for acc in (acc0, acc1):
        pltpu.sync_copy(nb_v.at[pl.ds(0, ZLEN)], acc.at[pl.ds(s * ZLEN, ZLEN)])
    plsc.subcore_barrier()

    for lb, acc in ((0, acc0), (1, acc1)):
        b = 2 * c + lb

        def chunk(j, _):
            e0 = s * EPT + j * CH
            pltpu.sync_copy(gp_hbm.at[b, pl.ds(e0 * 3, CH * 3)], gp_v)
            pltpu.sync_copy(w_hbm.at[b, pl.ds(e0, CH)], w_v)
            pltpu.sync_copy(idx_hbm.at[b, pl.ds(e0, CH)], idxc_v)

            def valq(q, _):
                e = 4 * q + sub
                gpv = plsc.load_gather(gp_v, [12 * q + gp_pat])
                wv = plsc.load_gather(w_v, [e])
                ei = plsc.load_gather(idxc_v, [e])
                val_v[pl.ds(16 * q, 16)] = jnp.where(d3, wv, gpv * wv)
                idx4_v[q >> 3, pl.ds(16 * (q & 7), 16)] = 4 * ei + d
                return _
            lax.fori_loop(0, NVAL, valq, None)

            copies = [
                pltpu.async_copy(val_v.at[pl.ds(t * GROUP, GROUP)],
                                 acc.at[idx4_v.at[t]], sem, add=True)
                for t in range(NGRP)
            ]
            for cp in copies:
                cp.wait()
            return _
        lax.fori_loop(0, NCHUNK, chunk, None)
    plsc.subcore_barrier()

    for lb, acc in ((0, acc0), (1, acc1)):
        b = 2 * c + lb
        pltpu.sync_copy(acc.at[pl.ds(s * ZLEN, ZLEN)], nb_v.at[pl.ds(0, ZLEN)])

        def outq(j, _):
            f = 16 * j + lane
            p = f // 3
            comp = f - 3 * p
            num = plsc.load_gather(nb_v, [4 * p + comp])
            den = plsc.load_gather(nb_v, [4 * p + 3])
            safe = jnp.where(den < 1e-10, one16, den)
            ob_v[pl.ds(16 * j, 16)] = num / safe
            return _
        lax.fori_loop(0, NOUT, outq, None)
        pltpu.sync_copy(ob_v, out_hbm.at[b, s])


@functools.partial(jax.jit, static_argnames=())
def _run(gp, w, idx):
    mesh = plsc.VectorSubcoreMesh(core_axis_name="c", subcore_axis_name="s",
                                  num_cores=NC, num_subcores=NT)
    f = pl.kernel(
        _sc_body,
        out_type=jax.ShapeDtypeStruct((B, NT, OPAD), jnp.float32),
        mesh=mesh,
        compiler_params=pltpu.CompilerParams(needs_layout_passes=False),
        scratch_types=[
            pltpu.VMEM((CH * 3,), jnp.float32),
            pltpu.VMEM((CH,), jnp.float32),
            pltpu.VMEM((CH,), jnp.int32),
            pltpu.VMEM((CH * 4,), jnp.float32),
            pltpu.VMEM((NGRP, GROUP), jnp.int32),
            pltpu.VMEM((NBL,), jnp.float32),
            pltpu.VMEM((OPAD,), jnp.float32),
            pltpu.VMEM_SHARED((NACC * 4,), jnp.float32),
            pltpu.VMEM_SHARED((NACC * 4,), jnp.float32),
            pltpu.SemaphoreType.DMA,
        ],
    )
    return f(gp, w, idx)


def kernel(grad_pred, x_t, grad_weight, Patchs_idx):
    del x_t
    gp = grad_pred.reshape(B, E * 3)
    w = grad_weight.reshape(B, E)
    idx = Patchs_idx.reshape(B, E)
    pad = EP - E
    gp = jnp.concatenate([gp, jnp.zeros((B, pad * 3), jnp.float32)], axis=1)
    w = jnp.concatenate([w, jnp.zeros((B, pad), jnp.float32)], axis=1)
    pad_idx = jnp.arange(pad, dtype=jnp.int32) % N
    idx = jnp.concatenate([idx, jnp.broadcast_to(pad_idx, (B, pad))], axis=1)
    out = _run(gp, w, idx)
    return out[:, :, :OPT].reshape(B, NACC, D)[:, :N]


# speedup vs baseline: 3.6919x; 1.2225x over previous
import functools

import jax
import jax.numpy as jnp
from jax import lax
from jax.experimental import pallas as pl
from jax.experimental.pallas import tpu as pltpu
from jax.experimental.pallas import tpu_sc as plsc

B, N, K, D = 4, 50000, 32, 3
E = N * K
NC, NT, L = 2, 16, 16
GROUP = 128
CH = 4096
CH_G = CH // GROUP
GTOT = E // GROUP
GPT = GTOT // NT
GXT = GTOT - GPT * NT
NCHUNK = GPT // CH_G
NPR = 3128
NACC = NPR * NT
ZLEN = NPR * 4
NBL = ZLEN + 32
OPT = NPR * D
OPAD = OPT + 8
NVAL = CH * 4 // L
NOUT = OPAD // L


def _sc_body(gp_hbm, w_hbm, idx_hbm, out_hbm,
             gp_v, w_v, idxc_v, val_v, idx4_v, valt_v, idx4t_v,
             nb_v, ob_v, acc0, acc1, sem):
    c = lax.axis_index("c")
    s = lax.axis_index("s")

    lane = lax.iota(jnp.int32, 16)
    sub = lane >> 2
    d = lane & 3
    d3 = d == 3
    gp_pat = 3 * sub + jnp.where(d3, 0, d)
    one16 = jnp.ones((16,), jnp.float32)

    zero16 = jnp.zeros((16,), jnp.float32)

    def zq(q, _):
        nb_v[pl.ds(16 * q, 16)] = zero16
        return _
    lax.fori_loop(0, ZLEN // 16, zq, None)
    for acc in (acc0, acc1):
        pltpu.sync_copy(nb_v.at[pl.ds(0, ZLEN)], acc.at[pl.ds(s * ZLEN, ZLEN)])
    plsc.subcore_barrier()

    g0 = GPT * s + jnp.minimum(s, GXT)
    ng = GPT + jnp.where(s < GXT, 1, 0)

    def do_chunk(acc, e0, ch, vv, iv):
        nval = ch * 4 // L
        pltpu.sync_copy(gp_hbm.at[pl.ds(e0 * 3, ch * 3)],
                        gp_v.at[pl.ds(0, ch * 3)])
        pltpu.sync_copy(w_hbm.at[pl.ds(e0, ch)], w_v.at[pl.ds(0, ch)])
        pltpu.sync_copy(idx_hbm.at[pl.ds(e0, ch)], idxc_v.at[pl.ds(0, ch)])

        def valq(q, _):
            e = 4 * q + sub
            gpv = plsc.load_gather(gp_v, [12 * q + gp_pat])
            wv = plsc.load_gather(w_v, [e])
            ei = plsc.load_gather(idxc_v, [e])
            vv[pl.ds(16 * q, 16)] = jnp.where(d3, wv, gpv * wv)
            iv[pl.ds(16 * q, 16)] = 4 * ei + d
            return _
        lax.fori_loop(0, nval, valq, None)

        pltpu.async_copy(vv, acc.at[iv], sem, add=True).wait()

    for lb, acc in ((0, acc0), (1, acc1)):
        b = 2 * c + lb
        ebase = b * E + g0 * GROUP

        def chunk(j, _):
            do_chunk(acc, ebase + j * CH, CH, val_v, idx4_v)
            return _
        lax.fori_loop(0, NCHUNK, chunk, None)

        def tail(t, _):
            do_chunk(acc, ebase + NCHUNK * CH + t * GROUP, GROUP,
                     valt_v, idx4t_v)
            return _
        lax.fori_loop(0, ng - NCHUNK * CH_G, tail, None)
    plsc.subcore_barrier()

    for lb, acc in ((0, acc0), (1, acc1)):
        b = 2 * c + lb
        pltpu.sync_copy(acc.at[pl.ds(s * ZLEN, ZLEN)], nb_v.at[pl.ds(0, ZLEN)])

        def outq(j, _):
            f = 16 * j + lane
            p = f // 3
            comp = f - 3 * p
            num = plsc.load_gather(nb_v, [4 * p + comp])
            den = plsc.load_gather(nb_v, [4 * p + 3])
            safe = jnp.where(den < 1e-10, one16, den)
            ob_v[pl.ds(16 * j, 16)] = num / safe
            return _
        lax.fori_loop(0, NOUT, outq, None)
        pltpu.sync_copy(ob_v, out_hbm.at[b, s])


@functools.partial(jax.jit, static_argnames=())
def _run(gp, w, idx):
    mesh = plsc.VectorSubcoreMesh(core_axis_name="c", subcore_axis_name="s",
                                  num_cores=NC, num_subcores=NT)
    f = pl.kernel(
        _sc_body,
        out_type=jax.ShapeDtypeStruct((B, NT, OPAD), jnp.float32),
        name="diffusion_sb2_scatter",
        mesh=mesh,
        compiler_params=pltpu.CompilerParams(needs_layout_passes=False),
        scratch_types=[
            pltpu.VMEM((CH * 3,), jnp.float32),
            pltpu.VMEM((CH,), jnp.float32),
            pltpu.VMEM((CH,), jnp.int32),
            pltpu.VMEM((CH * 4,), jnp.float32),
            pltpu.VMEM((CH * 4,), jnp.int32),
            pltpu.VMEM((GROUP * 4,), jnp.float32),
            pltpu.VMEM((GROUP * 4,), jnp.int32),
            pltpu.VMEM((NBL,), jnp.float32),
            pltpu.VMEM((OPAD,), jnp.float32),
            pltpu.VMEM_SHARED((NACC * 4,), jnp.float32),
            pltpu.VMEM_SHARED((NACC * 4,), jnp.float32),
            pltpu.SemaphoreType.DMA,
        ],
    )
    return f(gp, w, idx)


def kernel(grad_pred, x_t, grad_weight, Patchs_idx):
    del x_t
    gp = grad_pred.reshape(B * E * 3)
    w = grad_weight.reshape(B * E)
    idx = Patchs_idx.reshape(B * E)
    out = _run(gp, w, idx)
    return out[:, :, :OPT].reshape(B, NACC, D)[:, :N]


# speedup vs baseline: 49.1521x; 13.3135x over previous
import functools

import jax
import jax.numpy as jnp
from jax import lax
from jax.experimental import pallas as pl
from jax.experimental.pallas import tpu as pltpu
from jax.experimental.pallas import tpu_sc as plsc

B, N, K, D = 4, 50000, 32, 3
NC, NT, L = 2, 16, 16
NPAD = 50048
CHN = 8192
SPP = NPAD // CHN
NTAIL = NPAD - SPP * CHN
NSET = K * SPP
SETT = NSET // NT
NPR = 3128
NACC = NPR * NT
NBW = NPR + 8
OPT = NPR * D
OPAD = OPT + 8
NOUT = OPAD // L


def _sc_body(gp_hbm, w_hbm, idx_hbm, out_hbm,
             gx_v, gy_v, gz_v, w_v, idx_v, idxt_v,
             nbx, nby, nbz, nbw, ob_v,
             ax0, ay0, az0, aw0, ax1, ay1, az1, aw1, isem, ssem):
    c = lax.axis_index("c")
    s = lax.axis_index("s")

    lane = lax.iota(jnp.int32, 16)
    one16 = jnp.ones((16,), jnp.float32)
    zero16 = jnp.zeros((16,), jnp.float32)

    accs0 = (ax0, ay0, az0, aw0)
    accs1 = (ax1, ay1, az1, aw1)

    def zq(q, _):
        nbx[pl.ds(16 * q, 16)] = zero16
        return _
    lax.fori_loop(0, NBW // 16, zq, None)
    for accs in (accs0, accs1):
        for a in accs:
            pltpu.sync_copy(nbx.at[pl.ds(0, NPR)], a.at[pl.ds(s * NPR, NPR)])
    plsc.subcore_barrier()

    def do_set(accs, b, k, n0, nn, iv):
        ax, ay, az, aw = accs
        wbase = (b * K + k) * NPAD + n0
        cps = []
        for cc, buf in ((0, gx_v), (1, gy_v), (2, gz_v)):
            off = ((b * 3 + cc) * K + k) * NPAD + n0
            cps.append(pltpu.async_copy(gp_hbm.at[pl.ds(off, nn)],
                                        buf.at[pl.ds(0, nn)], isem))
        cps.append(pltpu.async_copy(w_hbm.at[pl.ds(wbase, nn)],
                                    w_v.at[pl.ds(0, nn)], isem))
        cps.append(pltpu.async_copy(idx_hbm.at[pl.ds(wbase, nn)],
                                    iv.at[pl.ds(0, nn)], isem))
        for cp in cps:
            cp.wait()

        @plsc.parallel_loop(0, nn // L, unroll=4)
        def _(m):
            wv = w_v[pl.ds(16 * m, 16)]
            gx_v[pl.ds(16 * m, 16)] = gx_v[pl.ds(16 * m, 16)] * wv
            gy_v[pl.ds(16 * m, 16)] = gy_v[pl.ds(16 * m, 16)] * wv
            gz_v[pl.ds(16 * m, 16)] = gz_v[pl.ds(16 * m, 16)] * wv

        cps = []
        for src, a in ((gx_v, ax), (gy_v, ay), (gz_v, az), (w_v, aw)):
            cps.append(pltpu.async_copy(src.at[pl.ds(0, nn)], a.at[iv],
                                        ssem, add=True))
        for cp in cps:
            cp.wait()

    for lb, accs in ((0, accs0), (1, accs1)):
        b = 2 * c + lb

        def mset(t, _):
            g = s + NT * t
            k = g // SPP
            n0 = (g - k * SPP) * CHN
            do_set(accs, b, k, n0, CHN, idx_v)
            return _
        lax.fori_loop(0, SETT, mset, None)

        for tk in range(2):
            do_set(accs, b, s + NT * tk, SPP * CHN, NTAIL, idxt_v)
    plsc.subcore_barrier()

    for lb, accs in ((0, accs0), (1, accs1)):
        b = 2 * c + lb
        ax, ay, az, aw = accs
        for a, nb in ((ax, nbx), (ay, nby), (az, nbz), (aw, nbw)):
            pltpu.sync_copy(a.at[pl.ds(s * NPR, NPR)], nb.at[pl.ds(0, NPR)])

        def outq(j, _):
            f = 16 * j + lane
            p = f // 3
            comp = f - 3 * p
            vx = plsc.load_gather(nbx, [p])
            vy = plsc.load_gather(nby, [p])
            vz = plsc.load_gather(nbz, [p])
            den = plsc.load_gather(nbw, [p])
            num = jnp.where(comp == 0, vx, jnp.where(comp == 1, vy, vz))
            safe = jnp.where(den < 1e-10, one16, den)
            ob_v[pl.ds(16 * j, 16)] = num / safe
            return _
        lax.fori_loop(0, NOUT, outq, None)
        pltpu.sync_copy(ob_v, out_hbm.at[b, s])


@functools.partial(jax.jit, static_argnames=())
def _run(gp, w, idx):
    mesh = plsc.VectorSubcoreMesh(core_axis_name="c", subcore_axis_name="s",
                                  num_cores=NC, num_subcores=NT)
    f = pl.kernel(
        _sc_body,
        out_type=jax.ShapeDtypeStruct((B, NT, OPAD), jnp.float32),
        name="diffusion_sb2_scatter",
        mesh=mesh,
        compiler_params=pltpu.CompilerParams(needs_layout_passes=False),
        scratch_types=[
            pltpu.VMEM((CHN,), jnp.float32),
            pltpu.VMEM((CHN,), jnp.float32),
            pltpu.VMEM((CHN,), jnp.float32),
            pltpu.VMEM((CHN,), jnp.float32),
            pltpu.VMEM((CHN,), jnp.int32),
            pltpu.VMEM((NTAIL,), jnp.int32),
            pltpu.VMEM((NBW,), jnp.float32),
            pltpu.VMEM((NBW,), jnp.float32),
            pltpu.VMEM((NBW,), jnp.float32),
            pltpu.VMEM((NBW,), jnp.float32),
            pltpu.VMEM((OPAD,), jnp.float32),
            pltpu.VMEM_SHARED((NACC,), jnp.float32),
            pltpu.VMEM_SHARED((NACC,), jnp.float32),
            pltpu.VMEM_SHARED((NACC,), jnp.float32),
            pltpu.VMEM_SHARED((NACC,), jnp.float32),
            pltpu.VMEM_SHARED((NACC,), jnp.float32),
            pltpu.VMEM_SHARED((NACC,), jnp.float32),
            pltpu.VMEM_SHARED((NACC,), jnp.float32),
            pltpu.VMEM_SHARED((NACC,), jnp.float32),
            pltpu.SemaphoreType.DMA,
            pltpu.SemaphoreType.DMA,
        ],
    )
    return f(gp, w, idx)


def kernel(grad_pred, x_t, grad_weight, Patchs_idx):
    del x_t
    pn = NPAD - N
    gp = jnp.pad(grad_pred, ((0, 0), (0, pn), (0, 0), (0, 0)))
    w = jnp.pad(grad_weight, ((0, 0), (0, pn), (0, 0), (0, 0)))
    idx = jnp.pad(Patchs_idx, ((0, 0), (0, pn), (0, 0)))
    gp = jnp.transpose(gp, (0, 3, 2, 1)).reshape(-1)
    w = jnp.transpose(w, (0, 2, 3, 1)).reshape(-1)
    idx = jnp.transpose(idx, (0, 2, 1)).reshape(-1)
    out = _run(gp, w, idx)
    return out[:, :, :OPT].reshape(B, NACC, D)[:, :N]


# speedup vs baseline: 52.9545x; 1.0774x over previous
import functools

import jax
import jax.numpy as jnp
from jax import lax
from jax.experimental import pallas as pl
from jax.experimental.pallas import tpu as pltpu
from jax.experimental.pallas import tpu_sc as plsc

B, N, K, D = 4, 50000, 32, 3
NC, NT, L = 2, 16, 16
NPAD = 50048
CHN = 8192
SPP = NPAD // CHN
NTAIL = NPAD - SPP * CHN
NSET = K * SPP
SETT = NSET // NT
NPR = 3128
NACC = NPR * NT
NBW = NPR + 8
OPT = NPR * D
OPAD = OPT + 8
NOUT = OPAD // L


def _sc_body(gp_hbm, w_hbm, idx_hbm, out_hbm,
             gxA, gyA, gzA, wA, iA, gxB, gyB, gzB, wB, iB, idxt_v,
             nbx, nby, nbz, nbw, ob_v,
             ax0, ay0, az0, aw0, ax1, ay1, az1, aw1,
             isemA, isemB, ssemA, ssemB, tsem):
    c = lax.axis_index("c")
    s = lax.axis_index("s")

    lane = lax.iota(jnp.int32, 16)
    one16 = jnp.ones((16,), jnp.float32)
    zero16 = jnp.zeros((16,), jnp.float32)

    accs0 = (ax0, ay0, az0, aw0)
    accs1 = (ax1, ay1, az1, aw1)

    def zq(q, _):
        nbx[pl.ds(16 * q, 16)] = zero16
        return _
    lax.fori_loop(0, NBW // 16, zq, None)
    for accs in (accs0, accs1):
        for a in accs:
            pltpu.sync_copy(nbx.at[pl.ds(0, NPR)], a.at[pl.ds(s * NPR, NPR)])
    plsc.subcore_barrier()

    bufsA = (gxA, gyA, gzA, wA, iA)
    bufsB = (gxB, gyB, gzB, wB, iB)

    def set_offsets(b, t):
        g = s + NT * t
        k = g // SPP
        n0 = (g - k * SPP) * CHN
        return k * NPAD + n0, (b * K + k) * NPAD + n0

    def fire_in(b, t, bufs, isem):
        kn0, wbase = set_offsets(b, t)
        gx, gy, gz, wb, ib = bufs
        for cc, buf in ((0, gx), (1, gy), (2, gz)):
            pltpu.async_copy(gp_hbm.at[pl.ds((b * 3 + cc) * K * NPAD + kn0,
                                             CHN)], buf, isem)
        pltpu.async_copy(w_hbm.at[pl.ds(wbase, CHN)], wb, isem)
        pltpu.async_copy(idx_hbm.at[pl.ds(wbase, CHN)], ib, isem)

    def wait_in(bufs, isem):
        gx, gy, gz, wb, ib = bufs
        for buf in (gx, gy, gz, wb):
            pltpu.make_async_copy(gp_hbm.at[pl.ds(0, CHN)], buf, isem).wait()
        pltpu.make_async_copy(idx_hbm.at[pl.ds(0, CHN)], ib, isem).wait()

    def compute(bufs):
        gx, gy, gz, wb, ib = bufs

        @plsc.parallel_loop(0, CHN // L, unroll=4)
        def _(m):
            wv = wb[pl.ds(16 * m, 16)]
            gx[pl.ds(16 * m, 16)] = gx[pl.ds(16 * m, 16)] * wv
            gy[pl.ds(16 * m, 16)] = gy[pl.ds(16 * m, 16)] * wv
            gz[pl.ds(16 * m, 16)] = gz[pl.ds(16 * m, 16)] * wv

    def fire_scat(accs, bufs, ssem):
        ax, ay, az, aw = accs
        gx, gy, gz, wb, ib = bufs
        for src, a in ((gx, ax), (gy, ay), (gz, az), (wb, aw)):
            pltpu.async_copy(src, a.at[ib], ssem, add=True)

    def wait_scat(accs, bufs, ssem):
        ax, ay, az, aw = accs
        gx, gy, gz, wb, ib = bufs
        for src, a in ((gx, ax), (gy, ay), (gz, az), (wb, aw)):
            pltpu.make_async_copy(src, a.at[ib], ssem).wait()

    for lb, accs in ((0, accs0), (1, accs1)):
        b = 2 * c + lb
        fire_in(b, 0, bufsA, isemA)

        def pair(jj, _):
            wait_in(bufsA, isemA)

            @pl.when(jj > 0)
            def _wB():
                wait_scat(accs, bufsB, ssemB)
            fire_in(b, 2 * jj + 1, bufsB, isemB)
            compute(bufsA)
            fire_scat(accs, bufsA, ssemA)

            wait_in(bufsB, isemB)

            @pl.when(jj < SETT // 2 - 1)
            def _fA():
                wait_scat(accs, bufsA, ssemA)
                fire_in(b, 2 * jj + 2, bufsA, isemA)
            compute(bufsB)
            fire_scat(accs, bufsB, ssemB)
            return _
        lax.fori_loop(0, SETT // 2, pair, None)
        wait_scat(accs, bufsA, ssemA)
        wait_scat(accs, bufsB, ssemB)

        ax, ay, az, aw = accs
        for tk in range(2):
            k = s + NT * tk
            wbase = (b * K + k) * NPAD + SPP * CHN
            cps = []
            for cc, buf in ((0, gxA), (1, gyA), (2, gzA)):
                off = ((b * 3 + cc) * K + k) * NPAD + SPP * CHN
                cps.append(pltpu.async_copy(gp_hbm.at[pl.ds(off, NTAIL)],
                                            buf.at[pl.ds(0, NTAIL)], tsem))
            cps.append(pltpu.async_copy(w_hbm.at[pl.ds(wbase, NTAIL)],
                                        wA.at[pl.ds(0, NTAIL)], tsem))
            cps.append(pltpu.async_copy(idx_hbm.at[pl.ds(wbase, NTAIL)],
                                        idxt_v, tsem))
            for cp in cps:
                cp.wait()

            @plsc.parallel_loop(0, NTAIL // L, unroll=4)
            def _(m):
                wv = wA[pl.ds(16 * m, 16)]
                gxA[pl.ds(16 * m, 16)] = gxA[pl.ds(16 * m, 16)] * wv
                gyA[pl.ds(16 * m, 16)] = gyA[pl.ds(16 * m, 16)] * wv
                gzA[pl.ds(16 * m, 16)] = gzA[pl.ds(16 * m, 16)] * wv

            cps = []
            for src, a in ((gxA, ax), (gyA, ay), (gzA, az), (wA, aw)):
                cps.append(pltpu.async_copy(src.at[pl.ds(0, NTAIL)],
                                            a.at[idxt_v], tsem, add=True))
            for cp in cps:
                cp.wait()
    plsc.subcore_barrier()

    for lb, accs in ((0, accs0), (1, accs1)):
        b = 2 * c + lb
        ax, ay, az, aw = accs
        for a, nb in ((ax, nbx), (ay, nby), (az, nbz), (aw, nbw)):
            pltpu.sync_copy(a.at[pl.ds(s * NPR, NPR)], nb.at[pl.ds(0, NPR)])

        def outq(j, _):
            f = 16 * j + lane
            p = f // 3
            comp = f - 3 * p
            vx = plsc.load_gather(nbx, [p])
            vy = plsc.load_gather(nby, [p])
            vz = plsc.load_gather(nbz, [p])
            den = plsc.load_gather(nbw, [p])
            num = jnp.where(comp == 0, vx, jnp.where(comp == 1, vy, vz))
            safe = jnp.where(den < 1e-10, one16, den)
            ob_v[pl.ds(16 * j, 16)] = num / safe
            return _
        lax.fori_loop(0, NOUT, outq, None)
        pltpu.sync_copy(ob_v, out_hbm.at[b, s])


@functools.partial(jax.jit, static_argnames=())
def _run(gp, w, idx):
    mesh = plsc.VectorSubcoreMesh(core_axis_name="c", subcore_axis_name="s",
                                  num_cores=NC, num_subcores=NT)
    f = pl.kernel(
        _sc_body,
        out_type=jax.ShapeDtypeStruct((B, NT, OPAD), jnp.float32),
        name="diffusion_sb2_scatter",
        mesh=mesh,
        compiler_params=pltpu.CompilerParams(needs_layout_passes=False),
        scratch_types=[
            pltpu.VMEM((CHN,), jnp.float32),
            pltpu.VMEM((CHN,), jnp.float32),
            pltpu.VMEM((CHN,), jnp.float32),
            pltpu.VMEM((CHN,), jnp.float32),
            pltpu.VMEM((CHN,), jnp.int32),
            pltpu.VMEM((CHN,), jnp.float32),
            pltpu.VMEM((CHN,), jnp.float32),
            pltpu.VMEM((CHN,), jnp.float32),
            pltpu.VMEM((CHN,), jnp.float32),
            pltpu.VMEM((CHN,), jnp.int32),
            pltpu.VMEM((NTAIL,), jnp.int32),
            pltpu.VMEM((NBW,), jnp.float32),
            pltpu.VMEM((NBW,), jnp.float32),
            pltpu.VMEM((NBW,), jnp.float32),
            pltpu.VMEM((NBW,), jnp.float32),
            pltpu.VMEM((OPAD,), jnp.float32),
            pltpu.VMEM_SHARED((NACC,), jnp.float32),
            pltpu.VMEM_SHARED((NACC,), jnp.float32),
            pltpu.VMEM_SHARED((NACC,), jnp.float32),
            pltpu.VMEM_SHARED((NACC,), jnp.float32),
            pltpu.VMEM_SHARED((NACC,), jnp.float32),
            pltpu.VMEM_SHARED((NACC,), jnp.float32),
            pltpu.VMEM_SHARED((NACC,), jnp.float32),
            pltpu.VMEM_SHARED((NACC,), jnp.float32),
            pltpu.SemaphoreType.DMA,
            pltpu.SemaphoreType.DMA,
            pltpu.SemaphoreType.DMA,
            pltpu.SemaphoreType.DMA,
            pltpu.SemaphoreType.DMA,
        ],
    )
    return f(gp, w, idx)


def kernel(grad_pred, x_t, grad_weight, Patchs_idx):
    del x_t
    pn = NPAD - N
    gp = jnp.pad(grad_pred, ((0, 0), (0, pn), (0, 0), (0, 0)))
    w = jnp.pad(grad_weight, ((0, 0), (0, pn), (0, 0), (0, 0)))
    idx = jnp.pad(Patchs_idx, ((0, 0), (0, pn), (0, 0)))
    gp = jnp.transpose(gp, (0, 3, 2, 1)).reshape(-1)
    w = jnp.transpose(w, (0, 2, 3, 1)).reshape(-1)
    idx = jnp.transpose(idx, (0, 2, 1)).reshape(-1)
    out = _run(gp, w, idx)
    return out[:, :, :OPT].reshape(B, NACC, D)[:, :N]


# speedup vs baseline: 75.1677x; 1.4195x over previous
import functools

import jax
import jax.numpy as jnp
from jax import lax
from jax.experimental import pallas as pl
from jax.experimental.pallas import tpu as pltpu
from jax.experimental.pallas import tpu_sc as plsc

B, N, K, D = 4, 50000, 32, 3
NC, NT, L = 2, 16, 16
NPAD = 50048
CHN = 8192
SPP = NPAD // CHN
NTAIL = NPAD - SPP * CHN
NSET = K * SPP
SETT = NSET // NT
NPR = 3128
NACC = NPR * NT
GALL = NPAD // 128
GBASE = 24
NBRD = (GBASE + 1) * 128
NACC2 = NACC + 128
OROW = 4 * 128


def _sc_body(gp_hbm, w_hbm, idx_hbm, out_hbm,
             gxA, gyA, gzA, wA, iA, gxB, gyB, gzB, wB, iB, idxt_v,
             nbx, nby, nbz, nbw,
             ax0, ay0, az0, aw0, ax1, ay1, az1, aw1,
             isemA, isemB, ssemA, ssemB, tsem, osem):
    c = lax.axis_index("c")
    s = lax.axis_index("s")

    lane = lax.iota(jnp.int32, 16)
    one16 = jnp.ones((16,), jnp.float32)
    zero16 = jnp.zeros((16,), jnp.float32)

    accs0 = (ax0, ay0, az0, aw0)
    accs1 = (ax1, ay1, az1, aw1)

    def zq(q, _):
        nbx[pl.ds(16 * q, 16)] = zero16
        return _
    lax.fori_loop(0, NBRD // 16, zq, None)
    for accs in (accs0, accs1):
        for a in accs:
            pltpu.sync_copy(nbx.at[pl.ds(0, NPR)], a.at[pl.ds(s * NPR, NPR)])
    plsc.subcore_barrier()

    bufsA = (gxA, gyA, gzA, wA, iA)
    bufsB = (gxB, gyB, gzB, wB, iB)

    def set_offsets(b, t):
        g = s + NT * t
        k = g // SPP
        n0 = (g - k * SPP) * CHN
        return k * NPAD + n0, (b * K + k) * NPAD + n0

    def fire_in(b, t, bufs, isem):
        kn0, wbase = set_offsets(b, t)
        gx, gy, gz, wb, ib = bufs
        for cc, buf in ((0, gx), (1, gy), (2, gz)):
            pltpu.async_copy(gp_hbm.at[pl.ds((b * 3 + cc) * K * NPAD + kn0,
                                             CHN)], buf, isem)
        pltpu.async_copy(w_hbm.at[pl.ds(wbase, CHN)], wb, isem)
        pltpu.async_copy(idx_hbm.at[pl.ds(wbase, CHN)], ib, isem)

    def wait_in(bufs, isem):
        gx, gy, gz, wb, ib = bufs
        for buf in (gx, gy, gz, wb):
            pltpu.make_async_copy(gp_hbm.at[pl.ds(0, CHN)], buf, isem).wait()
        pltpu.make_async_copy(idx_hbm.at[pl.ds(0, CHN)], ib, isem).wait()

    def compute(bufs):
        gx, gy, gz, wb, ib = bufs

        @plsc.parallel_loop(0, CHN // L, unroll=4)
        def _(m):
            wv = wb[pl.ds(16 * m, 16)]
            gx[pl.ds(16 * m, 16)] = gx[pl.ds(16 * m, 16)] * wv
            gy[pl.ds(16 * m, 16)] = gy[pl.ds(16 * m, 16)] * wv
            gz[pl.ds(16 * m, 16)] = gz[pl.ds(16 * m, 16)] * wv

    def fire_scat(accs, bufs, ssem):
        ax, ay, az, aw = accs
        gx, gy, gz, wb, ib = bufs
        for src, a in ((gx, ax), (gy, ay), (gz, az), (wb, aw)):
            pltpu.async_copy(src, a.at[ib], ssem, add=True)

    def wait_scat(accs, bufs, ssem):
        ax, ay, az, aw = accs
        gx, gy, gz, wb, ib = bufs
        for src, a in ((gx, ax), (gy, ay), (gz, az), (wb, aw)):
            pltpu.make_async_copy(src, a.at[ib], ssem).wait()

    for lb, accs in ((0, accs0), (1, accs1)):
        b = 2 * c + lb
        fire_in(b, 0, bufsA, isemA)

        def pair(jj, _):
            wait_in(bufsA, isemA)

            @pl.when(jj > 0)
            def _wB():
                wait_scat(accs, bufsB, ssemB)
            fire_in(b, 2 * jj + 1, bufsB, isemB)
            compute(bufsA)
            fire_scat(accs, bufsA, ssemA)

            wait_in(bufsB, isemB)

            @pl.when(jj < SETT // 2 - 1)
            def _fA():
                wait_scat(accs, bufsA, ssemA)
                fire_in(b, 2 * jj + 2, bufsA, isemA)
            compute(bufsB)
            fire_scat(accs, bufsB, ssemB)
            return _
        lax.fori_loop(0, SETT // 2, pair, None)
        wait_scat(accs, bufsA, ssemA)
        wait_scat(accs, bufsB, ssemB)

        ax, ay, az, aw = accs
        for tk in range(2):
            k = s + NT * tk
            wbase = (b * K + k) * NPAD + SPP * CHN
            cps = []
            for cc, buf in ((0, gxA), (1, gyA), (2, gzA)):
                off = ((b * 3 + cc) * K + k) * NPAD + SPP * CHN
                cps.append(pltpu.async_copy(gp_hbm.at[pl.ds(off, NTAIL)],
                                            buf.at[pl.ds(0, NTAIL)], tsem))
            cps.append(pltpu.async_copy(w_hbm.at[pl.ds(wbase, NTAIL)],
                                        wA.at[pl.ds(0, NTAIL)], tsem))
            cps.append(pltpu.async_copy(idx_hbm.at[pl.ds(wbase, NTAIL)],
                                        idxt_v, tsem))
            for cp in cps:
                cp.wait()

            @plsc.parallel_loop(0, NTAIL // L, unroll=4)
            def _(m):
                wv = wA[pl.ds(16 * m, 16)]
                gxA[pl.ds(16 * m, 16)] = gxA[pl.ds(16 * m, 16)] * wv
                gyA[pl.ds(16 * m, 16)] = gyA[pl.ds(16 * m, 16)] * wv
                gzA[pl.ds(16 * m, 16)] = gzA[pl.ds(16 * m, 16)] * wv

            cps = []
            for src, a in ((gxA, ax), (gyA, ay), (gzA, az), (wA, aw)):
                cps.append(pltpu.async_copy(src.at[pl.ds(0, NTAIL)],
                                            a.at[idxt_v], tsem, add=True))
            for cp in cps:
                cp.wait()
    plsc.subcore_barrier()

    g0 = GBASE * s + jnp.minimum(s, 7)
    ng = GBASE + jnp.where(s < 7, 1, 0)
    for lb, accs in ((0, accs0), (1, accs1)):
        b = 2 * c + lb
        ax, ay, az, aw = accs
        for a, nb in ((ax, nbx), (ay, nby), (az, nbz), (aw, nbw)):
            pltpu.sync_copy(a.at[pl.ds(g0 * 128, NBRD)], nb.at[pl.ds(0, NBRD)])

        @plsc.parallel_loop(0, NBRD // L, unroll=4)
        def _(m):
            den = nbw[pl.ds(16 * m, 16)]
            rec = one16 / jnp.where(den < 1e-10, one16, den)
            nbx[pl.ds(16 * m, 16)] = nbx[pl.ds(16 * m, 16)] * rec
            nby[pl.ds(16 * m, 16)] = nby[pl.ds(16 * m, 16)] * rec
            nbz[pl.ds(16 * m, 16)] = nbz[pl.ds(16 * m, 16)] * rec

        def wrow(t, _):
            obase = (g0 + t) * OROW + b * 128
            cps = [
                pltpu.async_copy(nb.at[pl.ds(128 * t, 128)],
                                 out_hbm.at[pl.ds(cc * (GALL * OROW) + obase,
                                                  128)], osem)
                for cc, nb in ((0, nbx), (1, nby), (2, nbz))
            ]
            for cp in cps:
                cp.wait()
            return _
        lax.fori_loop(0, ng, wrow, None)


@functools.partial(jax.jit, static_argnames=())
def _run(gp, w, idx):
    mesh = plsc.VectorSubcoreMesh(core_axis_name="c", subcore_axis_name="s",
                                  num_cores=NC, num_subcores=NT)
    f = pl.kernel(
        _sc_body,
        out_type=jax.ShapeDtypeStruct((D * GALL * OROW,), jnp.float32),
        name="diffusion_sb2_scatter",
        mesh=mesh,
        compiler_params=pltpu.CompilerParams(needs_layout_passes=False),
        scratch_types=[
            pltpu.VMEM((CHN,), jnp.float32),
            pltpu.VMEM((CHN,), jnp.float32),
            pltpu.VMEM((CHN,), jnp.float32),
            pltpu.VMEM((CHN,), jnp.float32),
            pltpu.VMEM((CHN,), jnp.int32),
            pltpu.VMEM((CHN,), jnp.float32),
            pltpu.VMEM((CHN,), jnp.float32),
            pltpu.VMEM((CHN,), jnp.float32),
            pltpu.VMEM((CHN,), jnp.float32),
            pltpu.VMEM((CHN,), jnp.int32),
            pltpu.VMEM((NTAIL,), jnp.int32),
            pltpu.VMEM((NBRD,), jnp.float32),
            pltpu.VMEM((NBRD,), jnp.float32),
            pltpu.VMEM((NBRD,), jnp.float32),
            pltpu.VMEM((NBRD,), jnp.float32),
            pltpu.VMEM_SHARED((NACC2,), jnp.float32),
            pltpu.VMEM_SHARED((NACC2,), jnp.float32),
            pltpu.VMEM_SHARED((NACC2,), jnp.float32),
            pltpu.VMEM_SHARED((NACC2,), jnp.float32),
            pltpu.VMEM_SHARED((NACC2,), jnp.float32),
            pltpu.VMEM_SHARED((NACC2,), jnp.float32),
            pltpu.VMEM_SHARED((NACC2,), jnp.float32),
            pltpu.VMEM_SHARED((NACC2,), jnp.float32),
            pltpu.SemaphoreType.DMA,
            pltpu.SemaphoreType.DMA,
            pltpu.SemaphoreType.DMA,
            pltpu.SemaphoreType.DMA,
            pltpu.SemaphoreType.DMA,
            pltpu.SemaphoreType.DMA,
        ],
    )
    return f(gp, w, idx)


def kernel(grad_pred, x_t, grad_weight, Patchs_idx):
    del x_t
    pn = NPAD - N
    gp = jnp.pad(grad_pred, ((0, 0), (0, pn), (0, 0), (0, 0)))
    w = jnp.pad(grad_weight, ((0, 0), (0, pn), (0, 0), (0, 0)))
    idx = jnp.pad(Patchs_idx, ((0, 0), (0, pn), (0, 0)))
    gp = jnp.transpose(gp, (0, 3, 2, 1)).reshape(-1)
    w = jnp.transpose(w, (0, 2, 3, 1)).reshape(-1)
    idx = jnp.transpose(idx, (0, 2, 1)).reshape(-1)
    out = _run(gp, w, idx)
    out = out.reshape(D, GALL, B, 128).transpose(2, 1, 3, 0)
    return out.reshape(B, NPAD, D)[:, :N]
